# baseline (speedup 1.0000x reference)
import functools

import jax
import jax.numpy as jnp
from jax import lax
from jax.experimental import pallas as pl
from jax.experimental.pallas import tpu as pltpu
from jax.experimental.pallas import tpu_sc as plsc

_N = 10000
_D = 128
_E = 320000
_NC = 2
_NS = 16
_NW = _NC * _NS
_CH = 64
_CPT = 160
_NCHUNK = _NW * _CPT
_EPAD = _NCHUNK * _CH
_NPAD = 10240
_RPT = _NPAD // _NS


def _mesh():
    return plsc.VectorSubcoreMesh(
        core_axis_name="c", subcore_axis_name="s",
        num_cores=_NC, num_subcores=_NS)


def _sc_degree(dstc, zvec, ones):
    @functools.partial(
        pl.kernel,
        out_type=jax.ShapeDtypeStruct((_NC * _NPAD,), jnp.float32),
        mesh=_mesh(),
        scratch_types=[
            pltpu.VMEM_SHARED((_NPAD,), jnp.float32),
            pltpu.VMEM((_CPT * _CH,), jnp.int32),
            pltpu.VMEM((_CPT * _CH,), jnp.float32),
            pltpu.VMEM((_RPT,), jnp.float32),
        ],
    )
    def k(dst_hbm, z_hbm, ones_hbm, out_hbm, deg_sh, idx_v, ones_v, buf_v):
        c = lax.axis_index("c")
        s = lax.axis_index("s")
        wid = s * _NC + c
        ept = _CPT * _CH
        pltpu.sync_copy(ones_hbm, ones_v)
        pltpu.sync_copy(dst_hbm.at[pl.ds(wid * ept, ept)], idx_v)
        pltpu.sync_copy(z_hbm, buf_v)
        pltpu.sync_copy(buf_v, deg_sh.at[pl.ds(s * _RPT, _RPT)])
        plsc.subcore_barrier()
        pltpu.sync_copy(ones_v, deg_sh.at[idx_v], add=True)
        plsc.subcore_barrier()
        pltpu.sync_copy(deg_sh.at[pl.ds(s * _RPT, _RPT)], buf_v)
        pltpu.sync_copy(buf_v, out_hbm.at[pl.ds(c * _NPAD + s * _RPT, _RPT)])

    return k(dstc, zvec, ones)


def _tc_prep(x, gamma, beta, W, degp):
    def body(x_ref, g_ref, be_ref, w_ref, deg_ref, hwp_ref, dis_ref):
        xv = x_ref[...]
        mean = jnp.mean(xv, axis=0, keepdims=True)
        xc = xv - mean
        var = jnp.mean(xc * xc, axis=0, keepdims=True)
        h = xc * lax.rsqrt(var + 1e-5) * g_ref[...] + be_ref[...]
        h = jnp.maximum(h, 0.0)
        hw = jnp.dot(h, w_ref[...], preferred_element_type=jnp.float32)
        deg = deg_ref[0] + deg_ref[1]
        dis = jnp.where(deg > 0.0, lax.rsqrt(jnp.maximum(deg, 1.0)), 0.0)
        dis_ref[...] = dis
        hwp_ref[...] = jnp.zeros((_NPAD, _D), jnp.float32)
        hwp_ref[pl.ds(0, _N), :] = hw * dis[: _N]

    return pl.pallas_call(
        body,
        out_shape=(
            jax.ShapeDtypeStruct((_NPAD, _D), jnp.float32),
            jax.ShapeDtypeStruct((_NPAD, 1), jnp.float32),
        ),
    )(x, gamma, beta, W, degp)


def _sc_agg(hwp, srcf, dstc, zmat):
    gch = 2 * _CH
    spr = 32
    gpr = spr // 2
    nr0 = 9
    nr1 = 2 * _CPT // spr - nr0

    @functools.partial(
        pl.kernel,
        out_type=jax.ShapeDtypeStruct((_NC, _NPAD, _D), jnp.float32),
        mesh=_mesh(),
        scratch_types=[
            pltpu.VMEM_SHARED((_NPAD, _D), jnp.float32),
            pltpu.VMEM((spr * _CH,), jnp.int32),
            pltpu.VMEM((spr, _CH), jnp.int32),
            pltpu.VMEM((2, gch, _D), jnp.float32),
            pltpu.SemaphoreType.DMA,
            pltpu.SemaphoreType.DMA,
        ],
    )
    def k(hwp_hbm, src_hbm, dst_hbm, z_hbm, out_hbm,
          agg_sh, idxs_v, idxd_v, rows_v, sem0, sem1):
        c = lax.axis_index("c")
        s = lax.axis_index("s")
        pltpu.sync_copy(z_hbm, rows_v.at[0, pl.ds(0, _CH)])

        def zb(r, carry):
            pltpu.sync_copy(rows_v.at[0, pl.ds(0, _CH)],
                            agg_sh.at[pl.ds(s * _RPT + r * _CH, _CH)])
            return carry

        lax.fori_loop(0, _RPT // _CH, zb, 0)
        plsc.subcore_barrier()

        def gather(g, buf, sem):
            pltpu.async_copy(hwp_hbm.at[idxs_v.at[pl.ds(g * gch, gch)]],
                             rows_v.at[buf], sem)

        def wait_scatter(g, k2, buf, sem):
            pltpu.make_async_copy(
                hwp_hbm.at[idxs_v.at[pl.ds(g * gch, gch)]],
                rows_v.at[buf], sem).wait()
            pltpu.sync_copy(rows_v.at[buf, pl.ds(0, _CH)],
                            agg_sh.at[idxd_v.at[k2]], add=True)
            pltpu.sync_copy(rows_v.at[buf, pl.ds(_CH, _CH)],
                            agg_sh.at[idxd_v.at[k2 + 1]], add=True)

        base = s * (_NC * _CPT) + jnp.where(c == 1, nr0 * spr, 0)
        nrounds = jnp.where(c == 1, nr1, nr0)

        def round_body(r, carry):
            cb = base + r * spr
            pltpu.sync_copy(src_hbm.at[pl.ds(cb * _CH, spr * _CH)], idxs_v)
            pltpu.sync_copy(dst_hbm.at[pl.ds(cb, spr)], idxd_v)
            gather(0, 0, sem0)

            def body(jo, carry2):
                ga = jo * 2
                gb = ga + 1
                gather(gb, 1, sem1)
                wait_scatter(ga, ga * 2, 0, sem0)

                @pl.when(jo + 1 < gpr // 2)
                def _():
                    gather(ga + 2, 0, sem0)

                wait_scatter(gb, gb * 2, 1, sem1)
                return carry2

            lax.fori_loop(0, gpr // 2, body, 0)
            return carry

        lax.fori_loop(0, nrounds, round_body, 0)
        plsc.subcore_barrier()

        def ob(r, carry):
            off = s * _RPT + r * _CH
            pltpu.sync_copy(agg_sh.at[pl.ds(off, _CH)],
                            rows_v.at[0, pl.ds(0, _CH)])
            pltpu.sync_copy(rows_v.at[0, pl.ds(0, _CH)],
                            out_hbm.at[c, pl.ds(off, _CH)])
            return carry

        lax.fori_loop(0, _RPT // _CH, ob, 0)

    return k(hwp, srcf, dstc, zmat)


def _tc_combine(x, aggp, dis, b):
    def body(x_ref, agg_ref, dis_ref, b_ref, o_ref):
        a = agg_ref[0, pl.ds(0, _N), :] + agg_ref[1, pl.ds(0, _N), :]
        o_ref[...] = x_ref[...] + a * dis_ref[pl.ds(0, _N), :] + b_ref[...]

    return pl.pallas_call(
        body,
        out_shape=jax.ShapeDtypeStruct((_N, _D), jnp.float32),
    )(x, aggp, dis, b)


def kernel(x, edge_index, gamma, beta, W, b):
    src = edge_index[0]
    dst = edge_index[1]
    pad = jnp.full((_EPAD - _E,), _N, dtype=jnp.int32)
    srcc = jnp.concatenate([src, pad]).reshape(_NCHUNK, _CH)
    dstc = jnp.concatenate([dst, pad]).reshape(_NCHUNK, _CH)
    zvec = jnp.zeros((_RPT,), jnp.float32)
    zmat = jnp.zeros((_CH, _D), jnp.float32)
    ones = jnp.ones((_CPT * _CH,), jnp.float32)

    degp = _sc_degree(dstc.reshape(-1), zvec, ones)
    hwp, dis = _tc_prep(x, gamma.reshape(1, _D), beta.reshape(1, _D), W,
                        degp.reshape(_NC, _NPAD, 1))
    aggp = _sc_agg(hwp, srcc.reshape(-1), dstc, zmat)
    return _tc_combine(x, aggp, dis, b.reshape(1, _D))


# speedup vs baseline: 1.0381x; 1.0381x over previous
import functools

import jax
import jax.numpy as jnp
from jax import lax
from jax.experimental import pallas as pl
from jax.experimental.pallas import tpu as pltpu
from jax.experimental.pallas import tpu_sc as plsc

_N = 10000
_D = 128
_E = 320000
_NC = 2
_NS = 16
_NW = _NC * _NS
_CH = 64
_CPT = 160
_NCHUNK = _NW * _CPT
_EPAD = _NCHUNK * _CH
_NPAD = 10240
_RPT = _NPAD // _NS


def _mesh():
    return plsc.VectorSubcoreMesh(
        core_axis_name="c", subcore_axis_name="s",
        num_cores=_NC, num_subcores=_NS)


def _sc_degree(dstc, zvec, ones):
    @functools.partial(
        pl.kernel,
        out_type=jax.ShapeDtypeStruct((_NC * _NPAD,), jnp.float32),
        mesh=_mesh(),
        scratch_types=[
            pltpu.VMEM_SHARED((_NPAD,), jnp.float32),
            pltpu.VMEM((_CPT * _CH,), jnp.int32),
            pltpu.VMEM((_CPT * _CH,), jnp.float32),
            pltpu.VMEM((_RPT,), jnp.float32),
        ],
    )
    def k(dst_hbm, z_hbm, ones_hbm, out_hbm, deg_sh, idx_v, ones_v, buf_v):
        c = lax.axis_index("c")
        s = lax.axis_index("s")
        wid = s * _NC + c
        ept = _CPT * _CH
        pltpu.sync_copy(ones_hbm, ones_v)
        pltpu.sync_copy(dst_hbm.at[pl.ds(wid * ept, ept)], idx_v)
        pltpu.sync_copy(z_hbm, buf_v)
        pltpu.sync_copy(buf_v, deg_sh.at[pl.ds(s * _RPT, _RPT)])
        plsc.subcore_barrier()
        pltpu.sync_copy(ones_v, deg_sh.at[idx_v], add=True)
        plsc.subcore_barrier()
        pltpu.sync_copy(deg_sh.at[pl.ds(s * _RPT, _RPT)], buf_v)
        pltpu.sync_copy(buf_v, out_hbm.at[pl.ds(c * _NPAD + s * _RPT, _RPT)])

    return k(dstc, zvec, ones)


def _tc_prep(x, gamma, beta, W, degp):
    def body(x_ref, g_ref, be_ref, w_ref, deg_ref, hwp_ref, dis_ref):
        xv = x_ref[...]
        mean = jnp.mean(xv, axis=0, keepdims=True)
        xc = xv - mean
        var = jnp.mean(xc * xc, axis=0, keepdims=True)
        h = xc * lax.rsqrt(var + 1e-5) * g_ref[...] + be_ref[...]
        h = jnp.maximum(h, 0.0)
        hw = jnp.dot(h, w_ref[...], preferred_element_type=jnp.float32)
        deg = deg_ref[0] + deg_ref[1]
        dis = jnp.where(deg > 0.0, lax.rsqrt(jnp.maximum(deg, 1.0)), 0.0)
        dis_ref[...] = dis
        hwp_ref[...] = jnp.zeros((_NPAD, _D), jnp.float32)
        hwp_ref[pl.ds(0, _N), :] = hw * dis[: _N]

    return pl.pallas_call(
        body,
        out_shape=(
            jax.ShapeDtypeStruct((_NPAD, _D), jnp.float32),
            jax.ShapeDtypeStruct((_NPAD, 1), jnp.float32),
        ),
    )(x, gamma, beta, W, degp)


def _sc_agg(hwp, srcf, dstc, zmat):
    gch = 2 * _CH
    spr = 32
    gpr = spr // 2
    nr0 = 8
    nr1 = 2 * _CPT // spr - nr0

    @functools.partial(
        pl.kernel,
        out_type=jax.ShapeDtypeStruct((_NC, _NPAD, _D), jnp.float32),
        mesh=_mesh(),
        scratch_types=[
            pltpu.VMEM_SHARED((_NPAD, _D), jnp.float32),
            pltpu.VMEM((spr * _CH,), jnp.int32),
            pltpu.VMEM((spr, _CH), jnp.int32),
            pltpu.VMEM((2, gch, _D), jnp.float32),
            pltpu.SemaphoreType.DMA,
            pltpu.SemaphoreType.DMA,
        ],
    )
    def k(hwp_hbm, src_hbm, dst_hbm, z_hbm, out_hbm,
          agg_sh, idxs_v, idxd_v, rows_v, sem0, sem1):
        c = lax.axis_index("c")
        s = lax.axis_index("s")
        pltpu.sync_copy(z_hbm, rows_v.at[0, pl.ds(0, _CH)])

        def zb(r, carry):
            pltpu.sync_copy(rows_v.at[0, pl.ds(0, _CH)],
                            agg_sh.at[pl.ds(s * _RPT + r * _CH, _CH)])
            return carry

        lax.fori_loop(0, _RPT // _CH, zb, 0)
        plsc.subcore_barrier()

        def gather(g, buf, sem):
            pltpu.async_copy(hwp_hbm.at[idxs_v.at[pl.ds(g * gch, gch)]],
                             rows_v.at[buf], sem)

        def wait_scatter(g, k2, buf, sem):
            pltpu.make_async_copy(
                hwp_hbm.at[idxs_v.at[pl.ds(g * gch, gch)]],
                rows_v.at[buf], sem).wait()
            pltpu.sync_copy(rows_v.at[buf, pl.ds(0, _CH)],
                            agg_sh.at[idxd_v.at[k2]], add=True)
            pltpu.sync_copy(rows_v.at[buf, pl.ds(_CH, _CH)],
                            agg_sh.at[idxd_v.at[k2 + 1]], add=True)

        base = s * (_NC * _CPT) + jnp.where(c == 1, nr0 * spr, 0)
        nrounds = jnp.where(c == 1, nr1, nr0)

        def round_body(r, carry):
            cb = base + r * spr
            pltpu.sync_copy(src_hbm.at[pl.ds(cb * _CH, spr * _CH)], idxs_v)
            pltpu.sync_copy(dst_hbm.at[pl.ds(cb, spr)], idxd_v)
            gather(0, 0, sem0)

            def body(jo, carry2):
                ga = jo * 2
                gb = ga + 1
                gather(gb, 1, sem1)
                wait_scatter(ga, ga * 2, 0, sem0)

                @pl.when(jo + 1 < gpr // 2)
                def _():
                    gather(ga + 2, 0, sem0)

                wait_scatter(gb, gb * 2, 1, sem1)
                return carry2

            lax.fori_loop(0, gpr // 2, body, 0)
            return carry

        lax.fori_loop(0, nrounds, round_body, 0)
        plsc.subcore_barrier()

        def ob(r, carry):
            off = s * _RPT + r * _CH
            pltpu.sync_copy(agg_sh.at[pl.ds(off, _CH)],
                            rows_v.at[0, pl.ds(0, _CH)])
            pltpu.sync_copy(rows_v.at[0, pl.ds(0, _CH)],
                            out_hbm.at[c, pl.ds(off, _CH)])
            return carry

        lax.fori_loop(0, _RPT // _CH, ob, 0)

    return k(hwp, srcf, dstc, zmat)


def _tc_combine(x, aggp, dis, b):
    def body(x_ref, agg_ref, dis_ref, b_ref, o_ref):
        a = agg_ref[0, pl.ds(0, _N), :] + agg_ref[1, pl.ds(0, _N), :]
        o_ref[...] = x_ref[...] + a * dis_ref[pl.ds(0, _N), :] + b_ref[...]

    return pl.pallas_call(
        body,
        out_shape=jax.ShapeDtypeStruct((_N, _D), jnp.float32),
    )(x, aggp, dis, b)


def kernel(x, edge_index, gamma, beta, W, b):
    src = edge_index[0]
    dst = edge_index[1]
    pad = jnp.full((_EPAD - _E,), _N, dtype=jnp.int32)
    srcc = jnp.concatenate([src, pad]).reshape(_NCHUNK, _CH)
    dstc = jnp.concatenate([dst, pad]).reshape(_NCHUNK, _CH)
    zvec = jnp.zeros((_RPT,), jnp.float32)
    zmat = jnp.zeros((_CH, _D), jnp.float32)
    ones = jnp.ones((_CPT * _CH,), jnp.float32)

    degp = _sc_degree(dstc.reshape(-1), zvec, ones)
    hwp, dis = _tc_prep(x, gamma.reshape(1, _D), beta.reshape(1, _D), W,
                        degp.reshape(_NC, _NPAD, 1))
    aggp = _sc_agg(hwp, srcc.reshape(-1), dstc, zmat)
    return _tc_combine(x, aggp, dis, b.reshape(1, _D))


# speedup vs baseline: 1.0471x; 1.0086x over previous
import functools

import jax
import jax.numpy as jnp
from jax import lax
from jax.experimental import pallas as pl
from jax.experimental.pallas import tpu as pltpu
from jax.experimental.pallas import tpu_sc as plsc

_N = 10000
_D = 128
_E = 320000
_NC = 2
_NS = 16
_NW = _NC * _NS
_CH = 64
_CPT = 160
_NCHUNK = _NW * _CPT
_EPAD = _NCHUNK * _CH
_NPAD = 10240
_RPT = _NPAD // _NS


def _mesh():
    return plsc.VectorSubcoreMesh(
        core_axis_name="c", subcore_axis_name="s",
        num_cores=_NC, num_subcores=_NS)


def _sc_degree(dstc, zvec, ones):
    @functools.partial(
        pl.kernel,
        out_type=jax.ShapeDtypeStruct((_NC * _NPAD,), jnp.float32),
        mesh=_mesh(),
        scratch_types=[
            pltpu.VMEM_SHARED((_NPAD,), jnp.float32),
            pltpu.VMEM((_CPT * _CH,), jnp.int32),
            pltpu.VMEM((_CPT * _CH,), jnp.float32),
            pltpu.VMEM((_RPT,), jnp.float32),
        ],
    )
    def k(dst_hbm, z_hbm, ones_hbm, out_hbm, deg_sh, idx_v, ones_v, buf_v):
        c = lax.axis_index("c")
        s = lax.axis_index("s")
        wid = s * _NC + c
        ept = _CPT * _CH
        pltpu.sync_copy(ones_hbm, ones_v)
        pltpu.sync_copy(dst_hbm.at[pl.ds(wid * ept, ept)], idx_v)
        pltpu.sync_copy(z_hbm, buf_v)
        pltpu.sync_copy(buf_v, deg_sh.at[pl.ds(s * _RPT, _RPT)])
        plsc.subcore_barrier()
        pltpu.sync_copy(ones_v, deg_sh.at[idx_v], add=True)
        plsc.subcore_barrier()
        pltpu.sync_copy(deg_sh.at[pl.ds(s * _RPT, _RPT)], buf_v)
        pltpu.sync_copy(buf_v, out_hbm.at[pl.ds(c * _NPAD + s * _RPT, _RPT)])

    return k(dstc, zvec, ones)


def _tc_prep(x, gamma, beta, W, degp):
    def body(x_ref, g_ref, be_ref, w_ref, deg_ref, hwp_ref, dis_ref):
        xv = x_ref[...]
        mean = jnp.mean(xv, axis=0, keepdims=True)
        xc = xv - mean
        var = jnp.mean(xc * xc, axis=0, keepdims=True)
        h = xc * lax.rsqrt(var + 1e-5) * g_ref[...] + be_ref[...]
        h = jnp.maximum(h, 0.0)
        hw = jnp.dot(h, w_ref[...], preferred_element_type=jnp.float32)
        deg = deg_ref[0] + deg_ref[1]
        dis = jnp.where(deg > 0.0, lax.rsqrt(jnp.maximum(deg, 1.0)), 0.0)
        dis_ref[...] = dis
        hwp_ref[...] = jnp.zeros((_NPAD, _D), jnp.float32)
        hwp_ref[pl.ds(0, _N), :] = hw * dis[: _N]

    return pl.pallas_call(
        body,
        out_shape=(
            jax.ShapeDtypeStruct((_NPAD, _D), jnp.float32),
            jax.ShapeDtypeStruct((_NPAD, 1), jnp.float32),
        ),
    )(x, gamma, beta, W, degp)


def _sc_agg(hwp, srcf, dstc, zmat):
    gch = 2 * _CH
    spr = 64
    gpr = spr // 2
    nr0 = 4
    nr1 = 2 * _CPT // spr - nr0

    @functools.partial(
        pl.kernel,
        out_type=jax.ShapeDtypeStruct((_NC, _NPAD, _D), jnp.float32),
        mesh=_mesh(),
        scratch_types=[
            pltpu.VMEM_SHARED((_NPAD, _D), jnp.float32),
            pltpu.VMEM((spr * _CH,), jnp.int32),
            pltpu.VMEM((spr, _CH), jnp.int32),
            pltpu.VMEM((2, gch, _D), jnp.float32),
            pltpu.SemaphoreType.DMA,
            pltpu.SemaphoreType.DMA,
        ],
    )
    def k(hwp_hbm, src_hbm, dst_hbm, z_hbm, out_hbm,
          agg_sh, idxs_v, idxd_v, rows_v, sem0, sem1):
        c = lax.axis_index("c")
        s = lax.axis_index("s")
        pltpu.sync_copy(z_hbm, rows_v.at[0, pl.ds(0, _CH)])

        def zb(r, carry):
            pltpu.sync_copy(rows_v.at[0, pl.ds(0, _CH)],
                            agg_sh.at[pl.ds(s * _RPT + r * _CH, _CH)])
            return carry

        lax.fori_loop(0, _RPT // _CH, zb, 0)
        plsc.subcore_barrier()

        def gather(g, buf, sem):
            pltpu.async_copy(hwp_hbm.at[idxs_v.at[pl.ds(g * gch, gch)]],
                             rows_v.at[buf], sem)

        def wait_scatter(g, k2, buf, sem):
            pltpu.make_async_copy(
                hwp_hbm.at[idxs_v.at[pl.ds(g * gch, gch)]],
                rows_v.at[buf], sem).wait()
            pltpu.sync_copy(rows_v.at[buf, pl.ds(0, _CH)],
                            agg_sh.at[idxd_v.at[k2]], add=True)
            pltpu.sync_copy(rows_v.at[buf, pl.ds(_CH, _CH)],
                            agg_sh.at[idxd_v.at[k2 + 1]], add=True)

        base = s * (_NC * _CPT) + jnp.where(c == 1, nr0 * spr, 0)
        nrounds = jnp.where(c == 1, nr1, nr0)

        def round_body(r, carry):
            cb = base + r * spr
            pltpu.sync_copy(src_hbm.at[pl.ds(cb * _CH, spr * _CH)], idxs_v)
            pltpu.sync_copy(dst_hbm.at[pl.ds(cb, spr)], idxd_v)
            gather(0, 0, sem0)

            def body(jo, carry2):
                ga = jo * 2
                gb = ga + 1
                gather(gb, 1, sem1)
                wait_scatter(ga, ga * 2, 0, sem0)

                @pl.when(jo + 1 < gpr // 2)
                def _():
                    gather(ga + 2, 0, sem0)

                wait_scatter(gb, gb * 2, 1, sem1)
                return carry2

            lax.fori_loop(0, gpr // 2, body, 0)
            return carry

        lax.fori_loop(0, nrounds, round_body, 0)
        plsc.subcore_barrier()

        def ob(r, carry):
            off = s * _RPT + r * _CH
            pltpu.sync_copy(agg_sh.at[pl.ds(off, _CH)],
                            rows_v.at[0, pl.ds(0, _CH)])
            pltpu.sync_copy(rows_v.at[0, pl.ds(0, _CH)],
                            out_hbm.at[c, pl.ds(off, _CH)])
            return carry

        lax.fori_loop(0, _RPT // _CH, ob, 0)

    return k(hwp, srcf, dstc, zmat)


def _tc_combine(x, aggp, dis, b):
    def body(x_ref, agg_ref, dis_ref, b_ref, o_ref):
        a = agg_ref[0, pl.ds(0, _N), :] + agg_ref[1, pl.ds(0, _N), :]
        o_ref[...] = x_ref[...] + a * dis_ref[pl.ds(0, _N), :] + b_ref[...]

    return pl.pallas_call(
        body,
        out_shape=jax.ShapeDtypeStruct((_N, _D), jnp.float32),
    )(x, aggp, dis, b)


def kernel(x, edge_index, gamma, beta, W, b):
    src = edge_index[0]
    dst = edge_index[1]
    pad = jnp.full((_EPAD - _E,), _N, dtype=jnp.int32)
    srcc = jnp.concatenate([src, pad]).reshape(_NCHUNK, _CH)
    dstc = jnp.concatenate([dst, pad]).reshape(_NCHUNK, _CH)
    zvec = jnp.zeros((_RPT,), jnp.float32)
    zmat = jnp.zeros((_CH, _D), jnp.float32)
    ones = jnp.ones((_CPT * _CH,), jnp.float32)

    degp = _sc_degree(dstc.reshape(-1), zvec, ones)
    hwp, dis = _tc_prep(x, gamma.reshape(1, _D), beta.reshape(1, _D), W,
                        degp.reshape(_NC, _NPAD, 1))
    aggp = _sc_agg(hwp, srcc.reshape(-1), dstc, zmat)
    return _tc_combine(x, aggp, dis, b.reshape(1, _D))
